# baseline (speedup 1.0000x reference)
import functools

import jax
import jax.numpy as jnp
import numpy as np
from jax import lax
from jax.experimental import pallas as pl
from jax.experimental.pallas import tpu as pltpu
from jax.experimental.pallas import tpu_sc as plsc

N = 10000
E = 160000
D = 128
A = 4

NODE_BLK = 2000
EDGE_BLK = 2000

SC_CORES = 2
SC_SUBCORES = 16
NW = SC_CORES * SC_SUBCORES
CH = 128
ROWS_PER_TILE = (N // SC_SUBCORES) // 8 * 8
ROWS_REMAINDER = N - SC_SUBCORES * ROWS_PER_TILE

_sc_mesh = plsc.VectorSubcoreMesh(core_axis_name="c", subcore_axis_name="s")


def _make_gather(n_edges):
    n_chunk = n_edges // CH
    base_ch = n_chunk // NW
    extra_ch = n_chunk % NW
    idxbuf = (base_ch + 1) * CH
    ngroup = (base_ch + 2) // 2

    @functools.partial(
        pl.kernel,
        out_type=(jax.ShapeDtypeStruct((n_edges, D), jnp.float32),
                  jax.ShapeDtypeStruct((n_edges, D), jnp.float32)),
        mesh=_sc_mesh,
        scratch_types=[
            pltpu.VMEM((idxbuf,), jnp.int32),
            pltpu.VMEM((idxbuf,), jnp.int32),
            pltpu.VMEM((2, CH, D), jnp.float32),
            pltpu.VMEM((2, CH, D), jnp.float32),
            pltpu.SemaphoreType.DMA,
            pltpu.SemaphoreType.DMA,
            pltpu.SemaphoreType.DMA,
            pltpu.SemaphoreType.DMA,
        ],
    )
    def gather2(h_hbm, dst_hbm, src_hbm, od_hbm, os_hbm,
                idxd_all, idxs_all, rowd, rows, sd0, sd1, ss0, ss1):
        wid = lax.axis_index("s") * SC_CORES + lax.axis_index("c")
        nch = base_ch + (wid < extra_ch).astype(jnp.int32)
        start = wid * base_ch + jnp.minimum(wid, extra_ch)
        base0 = start * CH
        pltpu.sync_copy(dst_hbm.at[pl.ds(base0, idxbuf)], idxd_all)
        pltpu.sync_copy(src_hbm.at[pl.ds(base0, idxbuf)], idxs_all)
        semd = (sd0, sd1)
        sems = (ss0, ss1)

        def fire(k, b):
            pltpu.async_copy(h_hbm.at[idxd_all.at[pl.ds(k * CH, CH)]],
                             rowd.at[b], semd[b])
            pltpu.async_copy(h_hbm.at[idxs_all.at[pl.ds(k * CH, CH)]],
                             rows.at[b], sems[b])

        fire(0, 0)
        fire(1, 1)

        def group(g, carry):
            for b in range(2):
                k = 2 * g + b

                @pl.when(k < nch)
                def _drain():
                    pltpu.make_async_copy(od_hbm.at[pl.ds(0, CH)],
                                          rowd.at[b], semd[b]).wait()
                    pltpu.make_async_copy(od_hbm.at[pl.ds(0, CH)],
                                          rows.at[b], sems[b]).wait()
                    pltpu.sync_copy(rowd.at[b],
                                    od_hbm.at[pl.ds(base0 + k * CH, CH)])
                    pltpu.sync_copy(rows.at[b],
                                    os_hbm.at[pl.ds(base0 + k * CH, CH)])

                    @pl.when(k + 2 < nch)
                    def _refill():
                        fire(k + 2, b)

            return carry

        lax.fori_loop(0, ngroup, group, 0)

    return gather2


def _make_scatter(n_edges):
    n_chunk = n_edges // CH
    base_ch = n_chunk // NW
    extra_ch = n_chunk % NW
    ngroup = (base_ch + 2) // 2

    @functools.partial(
        pl.kernel,
        out_type=jax.ShapeDtypeStruct((SC_CORES * N, D), jnp.float32),
        mesh=_sc_mesh,
        scratch_types=[
            pltpu.VMEM((CH,), jnp.int32),
            pltpu.VMEM((CH,), jnp.int32),
            pltpu.VMEM((2, CH, D), jnp.float32),
            pltpu.VMEM_SHARED((N, D), jnp.float32),
            pltpu.SemaphoreType.DMA,
            pltpu.SemaphoreType.DMA,
        ],
    )
    def scatter_add(m_hbm, dst_hbm, zeros_hbm, out_hbm,
                    idx0, idx1, rows, agg_sh, sm0, sm1):
        c = lax.axis_index("c")
        s = lax.axis_index("s")
        wid = s * SC_CORES + c
        nch = base_ch + (wid < extra_ch).astype(jnp.int32)
        start = wid * base_ch + jnp.minimum(wid, extra_ch)
        base0 = start * CH

        row0 = s * ROWS_PER_TILE
        pltpu.sync_copy(zeros_hbm.at[pl.ds(row0, ROWS_PER_TILE)],
                        agg_sh.at[pl.ds(row0, ROWS_PER_TILE)])

        @pl.when(s == SC_SUBCORES - 1)
        def _init_tail():
            tail0 = SC_SUBCORES * ROWS_PER_TILE
            pltpu.sync_copy(zeros_hbm.at[pl.ds(tail0, ROWS_REMAINDER)],
                            agg_sh.at[pl.ds(tail0, ROWS_REMAINDER)])

        plsc.subcore_barrier()

        sems = (sm0, sm1)
        idxs = (idx0, idx1)

        def fire(k, b):
            pltpu.async_copy(m_hbm.at[pl.ds(base0 + k * CH, CH)],
                             rows.at[b], sems[b])
            pltpu.sync_copy(dst_hbm.at[pl.ds(base0 + k * CH, CH)], idxs[b])

        fire(0, 0)
        fire(1, 1)

        def group(g, carry):
            for b in range(2):
                k = 2 * g + b

                @pl.when(k < nch)
                def _drain():
                    pltpu.make_async_copy(m_hbm.at[pl.ds(0, CH)],
                                          rows.at[b], sems[b]).wait()
                    pltpu.sync_copy(rows.at[b], agg_sh.at[idxs[b]], add=True)

                    @pl.when(k + 2 < nch)
                    def _refill():
                        fire(k + 2, b)

            return carry

        lax.fori_loop(0, ngroup, group, 0)
        plsc.subcore_barrier()
        pltpu.sync_copy(agg_sh.at[pl.ds(row0, ROWS_PER_TILE)],
                        out_hbm.at[pl.ds(c * N + row0, ROWS_PER_TILE)])

        @pl.when(s == SC_SUBCORES - 1)
        def _out_tail():
            tail0 = SC_SUBCORES * ROWS_PER_TILE
            pltpu.sync_copy(agg_sh.at[pl.ds(tail0, ROWS_REMAINDER)],
                            out_hbm.at[pl.ds(c * N + tail0, ROWS_REMAINDER)])

    return scatter_add


_gather_full = _make_gather(E)
_scatter_full = _make_scatter(E)


def _stack_w(W):
    scale = 1.0 / np.sqrt(W.shape[1] * W.shape[2])
    return jnp.transpose(W, (2, 1, 0)) * scale


def _silu(v):
    return v * jax.nn.sigmoid(v)


def _tp_sum(x, attr_cols, W_ref):
    acc = None
    for a in range(A):
        d = jnp.dot(x, W_ref[a], preferred_element_type=jnp.float32)
        d = d * attr_cols[a]
        acc = d if acc is None else acc + d
    return acc


def _two_stage_body(n_x2, silu_last, residual, edge_bf16, attr_as_cols):
    def body(*refs):
        xs, refs = refs[:1 + n_x2], refs[1 + n_x2:]
        if attr_as_cols:
            attr_refs, refs = refs[:A], refs[A:]
            attr = [r[...] for r in attr_refs]
        else:
            attr_ref, refs = refs[0], refs[1:]
            av = attr_ref[...]
            attr = [av[:, a : a + 1] for a in range(A)]
        Wa_ref, ba_ref, Wb_ref, bb_ref, out_ref = refs
        if n_x2 >= 2:
            x2 = xs[1][...]
            for r in xs[2:]:
                x2 = x2 + r[...]
            x = jnp.concatenate([xs[0][...], x2], axis=-1)
        elif n_x2 == 1:
            x = jnp.concatenate([xs[0][...], xs[1][...]], axis=-1)
        else:
            x = xs[0][...]
        if edge_bf16:
            x = x.astype(jnp.bfloat16)
        h1 = _silu(_tp_sum(x, attr, Wa_ref) + ba_ref[...])
        if edge_bf16:
            h1 = h1.astype(jnp.bfloat16)
        o = _tp_sum(h1, attr, Wb_ref) + bb_ref[...]
        if silu_last:
            o = _silu(o)
        if residual:
            o = o + xs[0][...]
        out_ref[...] = o

    return body


def _emb_body(x_ref, attr_ref, W_ref, b_ref, out_ref):
    av = attr_ref[...]
    attr = [av[:, a : a + 1] for a in range(A)]
    out_ref[...] = _tp_sum(x_ref[...], attr, W_ref) + b_ref[...]


def _full_spec(shape):
    nd = len(shape)
    return pl.BlockSpec(shape, lambda i, _n=nd: (0,) * _n)


def _tp2_call(x1, x2s, attr, Wa, ba, Wb, bb, *, silu_last, residual, blk,
              edge_bf16=False):
    M = x1.shape[0]
    assert M % blk == 0
    n_x2 = len(x2s)
    Was = _stack_w(Wa)
    Wbs = _stack_w(Wb)
    if edge_bf16:
        Was = Was.astype(jnp.bfloat16)
        Wbs = Wbs.astype(jnp.bfloat16)
    ba2 = ba.reshape(1, D)
    bb2 = bb.reshape(1, D)
    attr_as_cols = isinstance(attr, (list, tuple))
    attr_args = list(attr) if attr_as_cols else [attr]
    args = [x1] + [a for (a, _) in x2s] + attr_args + [Was, ba2, Wbs, bb2]
    in_specs = [pl.BlockSpec((blk, D), lambda i: (i, 0))]
    in_specs += [pl.BlockSpec((blk, D), lambda i, _o=off: (i + _o, 0))
                 for (_, off) in x2s]
    if attr_as_cols:
        in_specs += [pl.BlockSpec((blk, 1), lambda i: (i, 0))
                     for _ in range(A)]
    else:
        in_specs += [pl.BlockSpec((blk, A), lambda i: (i, 0))]
    in_specs += [
        _full_spec(Was.shape),
        _full_spec((1, D)),
        _full_spec(Wbs.shape),
        _full_spec((1, D)),
    ]
    return pl.pallas_call(
        _two_stage_body(n_x2, silu_last, residual, edge_bf16, attr_as_cols),
        grid=(M // blk,),
        in_specs=in_specs,
        out_specs=pl.BlockSpec((blk, D), lambda i: (i, 0)),
        out_shape=jax.ShapeDtypeStruct((M, D), jnp.float32),
    )(*args)


def _emb_call(x, attr, W, b, *, blk):
    M = x.shape[0]
    Ws = _stack_w(W)
    b2 = b.reshape(1, D)
    return pl.pallas_call(
        _emb_body,
        grid=(M // blk,),
        in_specs=[
            pl.BlockSpec((blk, D), lambda i: (i, 0)),
            pl.BlockSpec((blk, A), lambda i: (i, 0)),
            _full_spec(Ws.shape),
            _full_spec((1, D)),
        ],
        out_specs=pl.BlockSpec((blk, D), lambda i: (i, 0)),
        out_shape=jax.ShapeDtypeStruct((M, D), jnp.float32),
    )(x, attr, Ws, b2)


def kernel(x, pos, edge_index, edge_attr, node_attr, batch, W_emb, b_emb,
           W_msg1_0, b_msg1_0, W_msg2_0, b_msg2_0, W_upd1_0, b_upd1_0,
           W_upd2_0, b_upd2_0, W_msg1_1, b_msg1_1, W_msg2_1, b_msg2_1,
           W_upd1_1, b_upd1_1, W_upd2_1, b_upd2_1, W_pre1, b_pre1,
           W_pre2, b_pre2):
    na = node_attr.at[:, 0].set(1.0)
    h = _emb_call(x, na, W_emb, b_emb, blk=NODE_BLK)
    src = edge_index[0]
    dst = edge_index[1]
    pad = jnp.zeros((CH,), jnp.int32)
    dst_p = jnp.concatenate([dst, pad])
    src_p = jnp.concatenate([src, pad])
    ea_cols = [lax.slice(edge_attr, (0, a), (E, a + 1)) for a in range(A)]
    zeros_nd = jnp.zeros((N, D), jnp.float32)
    layers = [
        (W_msg1_0, b_msg1_0, W_msg2_0, b_msg2_0, W_upd1_0, b_upd1_0, W_upd2_0, b_upd2_0),
        (W_msg1_1, b_msg1_1, W_msg2_1, b_msg2_1, W_upd1_1, b_upd1_1, W_upd2_1, b_upd2_1),
    ]
    for (Wm1, bm1, Wm2, bm2, Wu1, bu1, Wu2, bu2) in layers:
        hd, hs = _gather_full(h, dst_p, src_p)
        m2 = _tp2_call(hd, [(hs, 0)], ea_cols, Wm1, bm1, Wm2, bm2,
                       silu_last=True, residual=False, blk=EDGE_BLK)
        agg2 = _scatter_full(m2, dst, zeros_nd)
        h = _tp2_call(h, [(agg2, 0), (agg2, N // NODE_BLK)], na,
                      Wu1, bu1, Wu2, bu2,
                      silu_last=False, residual=True, blk=NODE_BLK)
    h = _tp2_call(h, [], na, W_pre1, b_pre1, W_pre2, b_pre2,
                  silu_last=False, residual=False, blk=NODE_BLK)
    return h


# speedup vs baseline: 1.2086x; 1.2086x over previous
import functools

import jax
import jax.numpy as jnp
import numpy as np
from jax import lax
from jax.experimental import pallas as pl
from jax.experimental.pallas import tpu as pltpu
from jax.experimental.pallas import tpu_sc as plsc

N = 10000
E = 160000
D = 128
A = 4

NODE_BLK = 2000
EDGE_BLK = 2000

SC_CORES = 2
SC_SUBCORES = 16
NW = SC_CORES * SC_SUBCORES
CH = 128
ROWS_PER_TILE = (N // SC_SUBCORES) // 8 * 8
ROWS_REMAINDER = N - SC_SUBCORES * ROWS_PER_TILE

_sc_mesh = plsc.VectorSubcoreMesh(core_axis_name="c", subcore_axis_name="s")


def _make_gather(n_edges):
    n_chunk = n_edges // CH
    base_ch = n_chunk // NW
    extra_ch = n_chunk % NW
    idxbuf = (base_ch + 1) * CH
    ngroup = (base_ch + 2) // 2

    @functools.partial(
        pl.kernel,
        out_type=(jax.ShapeDtypeStruct((n_edges, D), jnp.float32),
                  jax.ShapeDtypeStruct((n_edges, D), jnp.float32)),
        mesh=_sc_mesh,
        scratch_types=[
            pltpu.VMEM((idxbuf,), jnp.int32),
            pltpu.VMEM((idxbuf,), jnp.int32),
            pltpu.VMEM((2, CH, D), jnp.float32),
            pltpu.VMEM((2, CH, D), jnp.float32),
            pltpu.SemaphoreType.DMA,
            pltpu.SemaphoreType.DMA,
            pltpu.SemaphoreType.DMA,
            pltpu.SemaphoreType.DMA,
        ],
    )
    def gather2(h_hbm, dst_hbm, src_hbm, od_hbm, os_hbm,
                idxd_all, idxs_all, rowd, rows, sd0, sd1, ss0, ss1):
        wid = lax.axis_index("s") * SC_CORES + lax.axis_index("c")
        nch = base_ch + (wid < extra_ch).astype(jnp.int32)
        start = wid * base_ch + jnp.minimum(wid, extra_ch)
        base0 = start * CH
        pltpu.sync_copy(dst_hbm.at[pl.ds(base0, idxbuf)], idxd_all)
        pltpu.sync_copy(src_hbm.at[pl.ds(base0, idxbuf)], idxs_all)
        semd = (sd0, sd1)
        sems = (ss0, ss1)

        def fire(k, b):
            pltpu.async_copy(h_hbm.at[idxd_all.at[pl.ds(k * CH, CH)]],
                             rowd.at[b], semd[b])
            pltpu.async_copy(h_hbm.at[idxs_all.at[pl.ds(k * CH, CH)]],
                             rows.at[b], sems[b])

        fire(0, 0)
        fire(1, 1)

        def group(g, carry):
            for b in range(2):
                k = 2 * g + b

                @pl.when(k < nch)
                def _drain():
                    pltpu.make_async_copy(od_hbm.at[pl.ds(0, CH)],
                                          rowd.at[b], semd[b]).wait()
                    pltpu.make_async_copy(od_hbm.at[pl.ds(0, CH)],
                                          rows.at[b], sems[b]).wait()
                    pltpu.sync_copy(rowd.at[b],
                                    od_hbm.at[pl.ds(base0 + k * CH, CH)])
                    pltpu.sync_copy(rows.at[b],
                                    os_hbm.at[pl.ds(base0 + k * CH, CH)])

                    @pl.when(k + 2 < nch)
                    def _refill():
                        fire(k + 2, b)

            return carry

        lax.fori_loop(0, ngroup, group, 0)

    return gather2


def _make_scatter(n_edges):
    n_chunk = n_edges // CH
    base_ch = n_chunk // NW
    extra_ch = n_chunk % NW
    ngroup = (base_ch + 2) // 2

    @functools.partial(
        pl.kernel,
        out_type=jax.ShapeDtypeStruct((SC_CORES * N, D), jnp.float32),
        mesh=_sc_mesh,
        scratch_types=[
            pltpu.VMEM((CH,), jnp.int32),
            pltpu.VMEM((CH,), jnp.int32),
            pltpu.VMEM((2, CH, D), jnp.float32),
            pltpu.VMEM_SHARED((N, D), jnp.float32),
            pltpu.SemaphoreType.DMA,
            pltpu.SemaphoreType.DMA,
        ],
    )
    def scatter_add(m_hbm, dst_hbm, zeros_hbm, out_hbm,
                    idx0, idx1, rows, agg_sh, sm0, sm1):
        c = lax.axis_index("c")
        s = lax.axis_index("s")
        wid = s * SC_CORES + c
        nch = base_ch + (wid < extra_ch).astype(jnp.int32)
        start = wid * base_ch + jnp.minimum(wid, extra_ch)
        base0 = start * CH

        row0 = s * ROWS_PER_TILE
        pltpu.sync_copy(zeros_hbm.at[pl.ds(row0, ROWS_PER_TILE)],
                        agg_sh.at[pl.ds(row0, ROWS_PER_TILE)])

        @pl.when(s == SC_SUBCORES - 1)
        def _init_tail():
            tail0 = SC_SUBCORES * ROWS_PER_TILE
            pltpu.sync_copy(zeros_hbm.at[pl.ds(tail0, ROWS_REMAINDER)],
                            agg_sh.at[pl.ds(tail0, ROWS_REMAINDER)])

        plsc.subcore_barrier()

        sems = (sm0, sm1)
        idxs = (idx0, idx1)

        def fire(k, b):
            pltpu.async_copy(m_hbm.at[pl.ds(base0 + k * CH, CH)],
                             rows.at[b], sems[b])
            pltpu.sync_copy(dst_hbm.at[pl.ds(base0 + k * CH, CH)], idxs[b])

        fire(0, 0)
        fire(1, 1)

        def group(g, carry):
            for b in range(2):
                k = 2 * g + b

                @pl.when(k < nch)
                def _drain():
                    pltpu.make_async_copy(m_hbm.at[pl.ds(0, CH)],
                                          rows.at[b], sems[b]).wait()
                    pltpu.sync_copy(rows.at[b], agg_sh.at[idxs[b]], add=True)

                    @pl.when(k + 2 < nch)
                    def _refill():
                        fire(k + 2, b)

            return carry

        lax.fori_loop(0, ngroup, group, 0)
        plsc.subcore_barrier()
        pltpu.sync_copy(agg_sh.at[pl.ds(row0, ROWS_PER_TILE)],
                        out_hbm.at[pl.ds(c * N + row0, ROWS_PER_TILE)])

        @pl.when(s == SC_SUBCORES - 1)
        def _out_tail():
            tail0 = SC_SUBCORES * ROWS_PER_TILE
            pltpu.sync_copy(agg_sh.at[pl.ds(tail0, ROWS_REMAINDER)],
                            out_hbm.at[pl.ds(c * N + tail0, ROWS_REMAINDER)])

    return scatter_add


_gather_full = _make_gather(E)
_scatter_full = _make_scatter(E)


def _stack_w(W):
    scale = 1.0 / np.sqrt(W.shape[1] * W.shape[2])
    return jnp.transpose(W, (2, 1, 0)) * scale


def _silu(v):
    return v * jax.nn.sigmoid(v)


def _tp_sum(x, attr_cols, W_ref):
    acc = None
    for a in range(A):
        d = jnp.dot(x, W_ref[a], preferred_element_type=jnp.float32)
        d = d * attr_cols[a]
        acc = d if acc is None else acc + d
    return acc


def _two_stage_body(n_x2, silu_last, residual, edge_bf16, attr_as_cols,
                    stage2_flat):
    def body(*refs):
        xs, refs = refs[:1 + n_x2], refs[1 + n_x2:]
        if attr_as_cols:
            attr_refs, refs = refs[:A], refs[A:]
            attr = [r[...] for r in attr_refs]
        else:
            attr_ref, refs = refs[0], refs[1:]
            av = attr_ref[...]
            attr = [av[:, a : a + 1] for a in range(A)]
        Wa_ref, ba_ref, Wb_ref, bb_ref, out_ref = refs
        if n_x2 >= 2:
            x2 = xs[1][...]
            for r in xs[2:]:
                x2 = x2 + r[...]
            x = jnp.concatenate([xs[0][...], x2], axis=-1)
        elif n_x2 == 1:
            x = jnp.concatenate([xs[0][...], xs[1][...]], axis=-1)
        else:
            x = xs[0][...]
        if edge_bf16:
            x = x.astype(jnp.bfloat16)
        h1 = _silu(_tp_sum(x, attr, Wa_ref) + ba_ref[...])
        if edge_bf16:
            h1 = h1.astype(jnp.bfloat16)
        if stage2_flat:
            y2 = jnp.concatenate([h1 * attr[a] for a in range(A)], axis=-1)
            o = jnp.dot(y2, Wb_ref[...],
                        preferred_element_type=jnp.float32) + bb_ref[...]
        else:
            o = _tp_sum(h1, attr, Wb_ref) + bb_ref[...]
        if silu_last:
            o = _silu(o)
        if residual:
            o = o + xs[0][...]
        out_ref[...] = o

    return body


def _emb_body(x_ref, attr_ref, W_ref, b_ref, out_ref):
    av = attr_ref[...]
    attr = [av[:, a : a + 1] for a in range(A)]
    out_ref[...] = _tp_sum(x_ref[...], attr, W_ref) + b_ref[...]


def _full_spec(shape):
    nd = len(shape)
    return pl.BlockSpec(shape, lambda i, _n=nd: (0,) * _n)


def _tp2_call(x1, x2s, attr, Wa, ba, Wb, bb, *, silu_last, residual, blk,
              edge_bf16=False, stage2_flat=False):
    M = x1.shape[0]
    assert M % blk == 0
    n_x2 = len(x2s)
    Was = _stack_w(Wa)
    Wbs = _stack_w(Wb)
    if stage2_flat:
        Wbs = Wbs.reshape(-1, D)
    if edge_bf16:
        Was = Was.astype(jnp.bfloat16)
        Wbs = Wbs.astype(jnp.bfloat16)
    ba2 = ba.reshape(1, D)
    bb2 = bb.reshape(1, D)
    attr_as_cols = isinstance(attr, (list, tuple))
    attr_args = list(attr) if attr_as_cols else [attr]
    args = [x1] + [a for (a, _) in x2s] + attr_args + [Was, ba2, Wbs, bb2]
    in_specs = [pl.BlockSpec((blk, D), lambda i: (i, 0))]
    in_specs += [pl.BlockSpec((blk, D), lambda i, _o=off: (i + _o, 0))
                 for (_, off) in x2s]
    if attr_as_cols:
        in_specs += [pl.BlockSpec((blk, 1), lambda i: (i, 0))
                     for _ in range(A)]
    else:
        in_specs += [pl.BlockSpec((blk, A), lambda i: (i, 0))]
    in_specs += [
        _full_spec(Was.shape),
        _full_spec((1, D)),
        _full_spec(Wbs.shape),
        _full_spec((1, D)),
    ]
    return pl.pallas_call(
        _two_stage_body(n_x2, silu_last, residual, edge_bf16, attr_as_cols,
                        stage2_flat),
        grid=(M // blk,),
        in_specs=in_specs,
        out_specs=pl.BlockSpec((blk, D), lambda i: (i, 0)),
        out_shape=jax.ShapeDtypeStruct((M, D), jnp.float32),
    )(*args)


def _emb_call(x, attr, W, b, *, blk):
    M = x.shape[0]
    Ws = _stack_w(W)
    b2 = b.reshape(1, D)
    return pl.pallas_call(
        _emb_body,
        grid=(M // blk,),
        in_specs=[
            pl.BlockSpec((blk, D), lambda i: (i, 0)),
            pl.BlockSpec((blk, A), lambda i: (i, 0)),
            _full_spec(Ws.shape),
            _full_spec((1, D)),
        ],
        out_specs=pl.BlockSpec((blk, D), lambda i: (i, 0)),
        out_shape=jax.ShapeDtypeStruct((M, D), jnp.float32),
    )(x, attr, Ws, b2)


def kernel(x, pos, edge_index, edge_attr, node_attr, batch, W_emb, b_emb,
           W_msg1_0, b_msg1_0, W_msg2_0, b_msg2_0, W_upd1_0, b_upd1_0,
           W_upd2_0, b_upd2_0, W_msg1_1, b_msg1_1, W_msg2_1, b_msg2_1,
           W_upd1_1, b_upd1_1, W_upd2_1, b_upd2_1, W_pre1, b_pre1,
           W_pre2, b_pre2):
    na = node_attr.at[:, 0].set(1.0)
    h = _emb_call(x, na, W_emb, b_emb, blk=NODE_BLK)
    src = edge_index[0]
    dst = edge_index[1]
    pad = jnp.zeros((CH,), jnp.int32)
    dst_p = jnp.concatenate([dst, pad])
    src_p = jnp.concatenate([src, pad])
    zeros_nd = jnp.zeros((N, D), jnp.float32)
    layers = [
        (W_msg1_0, b_msg1_0, W_msg2_0, b_msg2_0, W_upd1_0, b_upd1_0, W_upd2_0, b_upd2_0),
        (W_msg1_1, b_msg1_1, W_msg2_1, b_msg2_1, W_upd1_1, b_upd1_1, W_upd2_1, b_upd2_1),
    ]
    for (Wm1, bm1, Wm2, bm2, Wu1, bu1, Wu2, bu2) in layers:
        hd, hs = _gather_full(h, dst_p, src_p)
        m2 = _tp2_call(hd, [(hs, 0)], edge_attr, Wm1, bm1, Wm2, bm2,
                       silu_last=True, residual=False, blk=EDGE_BLK)
        agg2 = _scatter_full(m2, dst, zeros_nd)
        h = _tp2_call(h, [(agg2, 0), (agg2, N // NODE_BLK)], na,
                      Wu1, bu1, Wu2, bu2,
                      silu_last=False, residual=True, blk=NODE_BLK,
                      stage2_flat=True)
    h = _tp2_call(h, [], na, W_pre1, b_pre1, W_pre2, b_pre2,
                  silu_last=False, residual=False, blk=NODE_BLK,
                  stage2_flat=True)
    return h


# speedup vs baseline: 1.2482x; 1.0328x over previous
import functools

import jax
import jax.numpy as jnp
import numpy as np
from jax import lax
from jax.experimental import pallas as pl
from jax.experimental.pallas import tpu as pltpu
from jax.experimental.pallas import tpu_sc as plsc

N = 10000
E = 160000
D = 128
A = 4

NODE_BLK = 2000
EDGE_BLK = 2000

SC_CORES = 2
SC_SUBCORES = 16
NW = SC_CORES * SC_SUBCORES
CH = 128
ROWS_PER_TILE = (N // SC_SUBCORES) // 8 * 8
ROWS_REMAINDER = N - SC_SUBCORES * ROWS_PER_TILE

_sc_mesh = plsc.VectorSubcoreMesh(core_axis_name="c", subcore_axis_name="s")


def _make_gather(n_edges):
    n_chunk = n_edges // CH
    base_ch = n_chunk // NW
    extra_ch = n_chunk % NW
    idxbuf = (base_ch + 1) * CH
    nslot = 3
    ngroup = (base_ch + nslot) // nslot

    @functools.partial(
        pl.kernel,
        out_type=(jax.ShapeDtypeStruct((n_edges, D), jnp.float32),
                  jax.ShapeDtypeStruct((n_edges, D), jnp.float32)),
        mesh=_sc_mesh,
        scratch_types=[
            pltpu.VMEM((idxbuf,), jnp.int32),
            pltpu.VMEM((idxbuf,), jnp.int32),
            pltpu.VMEM((nslot, CH, D), jnp.float32),
            pltpu.VMEM((nslot, CH, D), jnp.float32),
        ] + [pltpu.SemaphoreType.DMA] * (2 * nslot),
    )
    def gather2(h_hbm, dst_hbm, src_hbm, od_hbm, os_hbm,
                idxd_all, idxs_all, rowd, rows, *sems_all):
        wid = lax.axis_index("s") * SC_CORES + lax.axis_index("c")
        nch = base_ch + (wid < extra_ch).astype(jnp.int32)
        start = wid * base_ch + jnp.minimum(wid, extra_ch)
        base0 = start * CH
        pltpu.sync_copy(dst_hbm.at[pl.ds(base0, idxbuf)], idxd_all)
        pltpu.sync_copy(src_hbm.at[pl.ds(base0, idxbuf)], idxs_all)
        semd = sems_all[:nslot]
        sems = sems_all[nslot:]

        def fire(k, b):
            pltpu.async_copy(h_hbm.at[idxd_all.at[pl.ds(k * CH, CH)]],
                             rowd.at[b], semd[b])
            pltpu.async_copy(h_hbm.at[idxs_all.at[pl.ds(k * CH, CH)]],
                             rows.at[b], sems[b])

        for b in range(nslot):
            fire(b, b)

        def group(g, carry):
            for b in range(nslot):
                k = nslot * g + b

                @pl.when(k < nch)
                def _drain():
                    pltpu.make_async_copy(od_hbm.at[pl.ds(0, CH)],
                                          rowd.at[b], semd[b]).wait()
                    pltpu.make_async_copy(od_hbm.at[pl.ds(0, CH)],
                                          rows.at[b], sems[b]).wait()
                    pltpu.sync_copy(rowd.at[b],
                                    od_hbm.at[pl.ds(base0 + k * CH, CH)])
                    pltpu.sync_copy(rows.at[b],
                                    os_hbm.at[pl.ds(base0 + k * CH, CH)])

                    @pl.when(k + nslot < nch)
                    def _refill():
                        fire(k + nslot, b)

            return carry

        lax.fori_loop(0, ngroup, group, 0)

    return gather2


def _make_scatter(n_edges):
    n_chunk = n_edges // CH
    base_ch = n_chunk // NW
    extra_ch = n_chunk % NW
    ngroup = (base_ch + 2) // 2

    @functools.partial(
        pl.kernel,
        out_type=jax.ShapeDtypeStruct((SC_CORES * N, D), jnp.float32),
        mesh=_sc_mesh,
        scratch_types=[
            pltpu.VMEM((CH,), jnp.int32),
            pltpu.VMEM((CH,), jnp.int32),
            pltpu.VMEM((2, CH, D), jnp.float32),
            pltpu.VMEM_SHARED((N, D), jnp.float32),
            pltpu.SemaphoreType.DMA,
            pltpu.SemaphoreType.DMA,
        ],
    )
    def scatter_add(m_hbm, dst_hbm, zeros_hbm, out_hbm,
                    idx0, idx1, rows, agg_sh, sm0, sm1):
        c = lax.axis_index("c")
        s = lax.axis_index("s")
        wid = s * SC_CORES + c
        nch = base_ch + (wid < extra_ch).astype(jnp.int32)
        start = wid * base_ch + jnp.minimum(wid, extra_ch)
        base0 = start * CH

        row0 = s * ROWS_PER_TILE
        pltpu.sync_copy(zeros_hbm.at[pl.ds(row0, ROWS_PER_TILE)],
                        agg_sh.at[pl.ds(row0, ROWS_PER_TILE)])

        @pl.when(s == SC_SUBCORES - 1)
        def _init_tail():
            tail0 = SC_SUBCORES * ROWS_PER_TILE
            pltpu.sync_copy(zeros_hbm.at[pl.ds(tail0, ROWS_REMAINDER)],
                            agg_sh.at[pl.ds(tail0, ROWS_REMAINDER)])

        plsc.subcore_barrier()

        sems = (sm0, sm1)
        idxs = (idx0, idx1)

        def fire(k, b):
            pltpu.async_copy(m_hbm.at[pl.ds(base0 + k * CH, CH)],
                             rows.at[b], sems[b])
            pltpu.sync_copy(dst_hbm.at[pl.ds(base0 + k * CH, CH)], idxs[b])

        fire(0, 0)
        fire(1, 1)

        def group(g, carry):
            for b in range(2):
                k = 2 * g + b

                @pl.when(k < nch)
                def _drain():
                    pltpu.make_async_copy(m_hbm.at[pl.ds(0, CH)],
                                          rows.at[b], sems[b]).wait()
                    pltpu.sync_copy(rows.at[b], agg_sh.at[idxs[b]], add=True)

                    @pl.when(k + 2 < nch)
                    def _refill():
                        fire(k + 2, b)

            return carry

        lax.fori_loop(0, ngroup, group, 0)
        plsc.subcore_barrier()
        pltpu.sync_copy(agg_sh.at[pl.ds(row0, ROWS_PER_TILE)],
                        out_hbm.at[pl.ds(c * N + row0, ROWS_PER_TILE)])

        @pl.when(s == SC_SUBCORES - 1)
        def _out_tail():
            tail0 = SC_SUBCORES * ROWS_PER_TILE
            pltpu.sync_copy(agg_sh.at[pl.ds(tail0, ROWS_REMAINDER)],
                            out_hbm.at[pl.ds(c * N + tail0, ROWS_REMAINDER)])

    return scatter_add


_gather_full = _make_gather(E)
_scatter_full = _make_scatter(E)


def _stack_w(W):
    scale = 1.0 / np.sqrt(W.shape[1] * W.shape[2])
    return jnp.transpose(W, (2, 1, 0)) * scale


def _silu(v):
    return v * jax.nn.sigmoid(v)


def _tp_sum(x, attr_cols, W_ref):
    acc = None
    for a in range(A):
        d = jnp.dot(x, W_ref[a], preferred_element_type=jnp.float32)
        d = d * attr_cols[a]
        acc = d if acc is None else acc + d
    return acc


def _two_stage_body(n_x2, silu_last, residual, edge_bf16, attr_as_cols,
                    stage2_flat):
    def body(*refs):
        xs, refs = refs[:1 + n_x2], refs[1 + n_x2:]
        if attr_as_cols:
            attr_refs, refs = refs[:A], refs[A:]
            attr = [r[...] for r in attr_refs]
        else:
            attr_ref, refs = refs[0], refs[1:]
            av = attr_ref[...]
            attr = [av[:, a : a + 1] for a in range(A)]
        Wa_ref, ba_ref, Wb_ref, bb_ref, out_ref = refs
        if n_x2 >= 2:
            x2 = xs[1][...]
            for r in xs[2:]:
                x2 = x2 + r[...]
            x = jnp.concatenate([xs[0][...], x2], axis=-1)
        elif n_x2 == 1:
            x = jnp.concatenate([xs[0][...], xs[1][...]], axis=-1)
        else:
            x = xs[0][...]
        if edge_bf16:
            x = x.astype(jnp.bfloat16)
        h1 = _silu(_tp_sum(x, attr, Wa_ref) + ba_ref[...])
        if edge_bf16:
            h1 = h1.astype(jnp.bfloat16)
        if stage2_flat:
            y2 = jnp.concatenate([h1 * attr[a] for a in range(A)], axis=-1)
            o = jnp.dot(y2, Wb_ref[...],
                        preferred_element_type=jnp.float32) + bb_ref[...]
        else:
            o = _tp_sum(h1, attr, Wb_ref) + bb_ref[...]
        if silu_last:
            o = _silu(o)
        if residual:
            o = o + xs[0][...]
        out_ref[...] = o

    return body


def _emb_body(x_ref, attr_ref, W_ref, b_ref, out_ref):
    av = attr_ref[...]
    attr = [av[:, a : a + 1] for a in range(A)]
    out_ref[...] = _tp_sum(x_ref[...], attr, W_ref) + b_ref[...]


def _full_spec(shape):
    nd = len(shape)
    return pl.BlockSpec(shape, lambda i, _n=nd: (0,) * _n)


def _tp2_call(x1, x2s, attr, Wa, ba, Wb, bb, *, silu_last, residual, blk,
              edge_bf16=False, stage2_flat=False):
    M = x1.shape[0]
    assert M % blk == 0
    n_x2 = len(x2s)
    Was = _stack_w(Wa)
    Wbs = _stack_w(Wb)
    if stage2_flat:
        Wbs = Wbs.reshape(-1, D)
    if edge_bf16:
        Was = Was.astype(jnp.bfloat16)
        Wbs = Wbs.astype(jnp.bfloat16)
    ba2 = ba.reshape(1, D)
    bb2 = bb.reshape(1, D)
    attr_as_cols = isinstance(attr, (list, tuple))
    attr_args = list(attr) if attr_as_cols else [attr]
    args = [x1] + [a for (a, _) in x2s] + attr_args + [Was, ba2, Wbs, bb2]
    in_specs = [pl.BlockSpec((blk, D), lambda i: (i, 0))]
    in_specs += [pl.BlockSpec((blk, D), lambda i, _o=off: (i + _o, 0))
                 for (_, off) in x2s]
    if attr_as_cols:
        in_specs += [pl.BlockSpec((blk, 1), lambda i: (i, 0))
                     for _ in range(A)]
    else:
        in_specs += [pl.BlockSpec((blk, A), lambda i: (i, 0))]
    in_specs += [
        _full_spec(Was.shape),
        _full_spec((1, D)),
        _full_spec(Wbs.shape),
        _full_spec((1, D)),
    ]
    return pl.pallas_call(
        _two_stage_body(n_x2, silu_last, residual, edge_bf16, attr_as_cols,
                        stage2_flat),
        grid=(M // blk,),
        in_specs=in_specs,
        out_specs=pl.BlockSpec((blk, D), lambda i: (i, 0)),
        out_shape=jax.ShapeDtypeStruct((M, D), jnp.float32),
    )(*args)


def _emb_call(x, attr, W, b, *, blk):
    M = x.shape[0]
    Ws = _stack_w(W)
    b2 = b.reshape(1, D)
    return pl.pallas_call(
        _emb_body,
        grid=(M // blk,),
        in_specs=[
            pl.BlockSpec((blk, D), lambda i: (i, 0)),
            pl.BlockSpec((blk, A), lambda i: (i, 0)),
            _full_spec(Ws.shape),
            _full_spec((1, D)),
        ],
        out_specs=pl.BlockSpec((blk, D), lambda i: (i, 0)),
        out_shape=jax.ShapeDtypeStruct((M, D), jnp.float32),
    )(x, attr, Ws, b2)


def kernel(x, pos, edge_index, edge_attr, node_attr, batch, W_emb, b_emb,
           W_msg1_0, b_msg1_0, W_msg2_0, b_msg2_0, W_upd1_0, b_upd1_0,
           W_upd2_0, b_upd2_0, W_msg1_1, b_msg1_1, W_msg2_1, b_msg2_1,
           W_upd1_1, b_upd1_1, W_upd2_1, b_upd2_1, W_pre1, b_pre1,
           W_pre2, b_pre2):
    na = jnp.where(jnp.arange(A) == 0, 1.0, node_attr)
    h = _emb_call(x, na, W_emb, b_emb, blk=NODE_BLK)
    src = edge_index[0]
    dst = edge_index[1]
    pad = jnp.zeros((CH,), jnp.int32)
    dst_p = jnp.concatenate([dst, pad])
    src_p = jnp.concatenate([src, pad])
    zeros_nd = jnp.zeros((N, D), jnp.float32)
    layers = [
        (W_msg1_0, b_msg1_0, W_msg2_0, b_msg2_0, W_upd1_0, b_upd1_0, W_upd2_0, b_upd2_0),
        (W_msg1_1, b_msg1_1, W_msg2_1, b_msg2_1, W_upd1_1, b_upd1_1, W_upd2_1, b_upd2_1),
    ]
    for (Wm1, bm1, Wm2, bm2, Wu1, bu1, Wu2, bu2) in layers:
        hd, hs = _gather_full(h, dst_p, src_p)
        m2 = _tp2_call(hd, [(hs, 0)], edge_attr, Wm1, bm1, Wm2, bm2,
                       silu_last=True, residual=False, blk=EDGE_BLK)
        agg2 = _scatter_full(m2, dst, zeros_nd)
        h = _tp2_call(h, [(agg2, 0), (agg2, N // NODE_BLK)], na,
                      Wu1, bu1, Wu2, bu2,
                      silu_last=False, residual=True, blk=NODE_BLK,
                      stage2_flat=True)
    h = _tp2_call(h, [], na, W_pre1, b_pre1, W_pre2, b_pre2,
                  silu_last=False, residual=False, blk=NODE_BLK,
                  stage2_flat=True)
    return h


# speedup vs baseline: 1.2997x; 1.0413x over previous
import functools

import jax
import jax.numpy as jnp
import numpy as np
from jax import lax
from jax.experimental import pallas as pl
from jax.experimental.pallas import tpu as pltpu
from jax.experimental.pallas import tpu_sc as plsc

N = 10000
E = 160000
D = 128
A = 4

NODE_BLK = 2000
EDGE_BLK = 2000

SC_CORES = 2
SC_SUBCORES = 16
NW = SC_CORES * SC_SUBCORES
CH = 128
ROWS_PER_TILE = (N // SC_SUBCORES) // 8 * 8
ROWS_REMAINDER = N - SC_SUBCORES * ROWS_PER_TILE

_sc_mesh = plsc.VectorSubcoreMesh(core_axis_name="c", subcore_axis_name="s")


def _make_gather(n_edges):
    n_chunk = n_edges // CH
    base_ch = n_chunk // NW
    extra_ch = n_chunk % NW
    idxbuf = (base_ch + 1) * CH
    nslot = 3
    ngroup = (base_ch + nslot) // nslot

    @functools.partial(
        pl.kernel,
        out_type=(jax.ShapeDtypeStruct((n_edges, D), jnp.float32),
                  jax.ShapeDtypeStruct((n_edges, D), jnp.float32)),
        mesh=_sc_mesh,
        scratch_types=[
            pltpu.VMEM((idxbuf,), jnp.int32),
            pltpu.VMEM((idxbuf,), jnp.int32),
            pltpu.VMEM((nslot, CH, D), jnp.float32),
            pltpu.VMEM((nslot, CH, D), jnp.float32),
        ] + [pltpu.SemaphoreType.DMA] * (2 * nslot),
    )
    def gather2(h_hbm, dst_hbm, src_hbm, od_hbm, os_hbm,
                idxd_all, idxs_all, rowd, rows, *sems_all):
        wid = lax.axis_index("s") * SC_CORES + lax.axis_index("c")
        nch = base_ch + (wid < extra_ch).astype(jnp.int32)
        start = wid * base_ch + jnp.minimum(wid, extra_ch)
        base0 = start * CH
        pltpu.sync_copy(dst_hbm.at[pl.ds(base0, idxbuf)], idxd_all)
        pltpu.sync_copy(src_hbm.at[pl.ds(base0, idxbuf)], idxs_all)
        semd = sems_all[:nslot]
        sems = sems_all[nslot:]

        def fire(k, b):
            pltpu.async_copy(h_hbm.at[idxd_all.at[pl.ds(k * CH, CH)]],
                             rowd.at[b], semd[b])
            pltpu.async_copy(h_hbm.at[idxs_all.at[pl.ds(k * CH, CH)]],
                             rows.at[b], sems[b])

        for b in range(nslot):
            fire(b, b)

        def group(g, carry):
            for b in range(nslot):
                k = nslot * g + b

                @pl.when(k < nch)
                def _drain():
                    pltpu.make_async_copy(od_hbm.at[pl.ds(0, CH)],
                                          rowd.at[b], semd[b]).wait()
                    pltpu.make_async_copy(od_hbm.at[pl.ds(0, CH)],
                                          rows.at[b], sems[b]).wait()
                    pltpu.sync_copy(rowd.at[b],
                                    od_hbm.at[pl.ds(base0 + k * CH, CH)])
                    pltpu.sync_copy(rows.at[b],
                                    os_hbm.at[pl.ds(base0 + k * CH, CH)])

                    @pl.when(k + nslot < nch)
                    def _refill():
                        fire(k + nslot, b)

            return carry

        lax.fori_loop(0, ngroup, group, 0)

    return gather2


def _make_scatter2(n_edges, n_a):
    n_chunk = n_edges // CH
    ch_a = n_a // CH
    base_ch = n_chunk // NW
    extra_ch = n_chunk % NW
    ngroup = (base_ch + 2) // 2

    @functools.partial(
        pl.kernel,
        out_type=jax.ShapeDtypeStruct((SC_CORES * N, D), jnp.float32),
        mesh=_sc_mesh,
        scratch_types=[
            pltpu.VMEM((CH,), jnp.int32),
            pltpu.VMEM((CH,), jnp.int32),
            pltpu.VMEM((2, CH, D), jnp.float32),
            pltpu.VMEM_SHARED((N, D), jnp.float32),
            pltpu.SemaphoreType.DMA,
            pltpu.SemaphoreType.DMA,
        ],
    )
    def scatter_add2(ma_hbm, mb_hbm, dst_hbm, zeros_hbm, out_hbm,
                     idx0, idx1, rows, agg_sh, sm0, sm1):
        c = lax.axis_index("c")
        s = lax.axis_index("s")
        wid = s * SC_CORES + c
        nch = base_ch + (wid < extra_ch).astype(jnp.int32)
        start = wid * base_ch + jnp.minimum(wid, extra_ch)
        base0 = start * CH

        row0 = s * ROWS_PER_TILE
        pltpu.sync_copy(zeros_hbm.at[pl.ds(row0, ROWS_PER_TILE)],
                        agg_sh.at[pl.ds(row0, ROWS_PER_TILE)])

        @pl.when(s == SC_SUBCORES - 1)
        def _init_tail():
            tail0 = SC_SUBCORES * ROWS_PER_TILE
            pltpu.sync_copy(zeros_hbm.at[pl.ds(tail0, ROWS_REMAINDER)],
                            agg_sh.at[pl.ds(tail0, ROWS_REMAINDER)])

        plsc.subcore_barrier()

        sems = (sm0, sm1)
        idxs = (idx0, idx1)

        def fire(k, b):
            kg = start + k

            @pl.when(kg < ch_a)
            def _fa():
                pltpu.async_copy(ma_hbm.at[pl.ds(kg * CH, CH)],
                                 rows.at[b], sems[b])

            @pl.when(kg >= ch_a)
            def _fb():
                pltpu.async_copy(mb_hbm.at[pl.ds((kg - ch_a) * CH, CH)],
                                 rows.at[b], sems[b])

            pltpu.sync_copy(dst_hbm.at[pl.ds(base0 + k * CH, CH)], idxs[b])

        fire(0, 0)
        fire(1, 1)

        def group(g, carry):
            for b in range(2):
                k = 2 * g + b

                @pl.when(k < nch)
                def _drain():
                    pltpu.make_async_copy(ma_hbm.at[pl.ds(0, CH)],
                                          rows.at[b], sems[b]).wait()
                    pltpu.sync_copy(rows.at[b], agg_sh.at[idxs[b]], add=True)

                    @pl.when(k + 2 < nch)
                    def _refill():
                        fire(k + 2, b)

            return carry

        lax.fori_loop(0, ngroup, group, 0)
        plsc.subcore_barrier()
        pltpu.sync_copy(agg_sh.at[pl.ds(row0, ROWS_PER_TILE)],
                        out_hbm.at[pl.ds(c * N + row0, ROWS_PER_TILE)])

        @pl.when(s == SC_SUBCORES - 1)
        def _out_tail():
            tail0 = SC_SUBCORES * ROWS_PER_TILE
            pltpu.sync_copy(agg_sh.at[pl.ds(tail0, ROWS_REMAINDER)],
                            out_hbm.at[pl.ds(c * N + tail0, ROWS_REMAINDER)])

    return scatter_add2


def _make_scatter(n_edges):
    n_chunk = n_edges // CH
    base_ch = n_chunk // NW
    extra_ch = n_chunk % NW
    ngroup = (base_ch + 2) // 2

    @functools.partial(
        pl.kernel,
        out_type=jax.ShapeDtypeStruct((SC_CORES * N, D), jnp.float32),
        mesh=_sc_mesh,
        scratch_types=[
            pltpu.VMEM((CH,), jnp.int32),
            pltpu.VMEM((CH,), jnp.int32),
            pltpu.VMEM((2, CH, D), jnp.float32),
            pltpu.VMEM_SHARED((N, D), jnp.float32),
            pltpu.SemaphoreType.DMA,
            pltpu.SemaphoreType.DMA,
        ],
    )
    def scatter_add(m_hbm, dst_hbm, zeros_hbm, out_hbm,
                    idx0, idx1, rows, agg_sh, sm0, sm1):
        c = lax.axis_index("c")
        s = lax.axis_index("s")
        wid = s * SC_CORES + c
        nch = base_ch + (wid < extra_ch).astype(jnp.int32)
        start = wid * base_ch + jnp.minimum(wid, extra_ch)
        base0 = start * CH

        row0 = s * ROWS_PER_TILE
        pltpu.sync_copy(zeros_hbm.at[pl.ds(row0, ROWS_PER_TILE)],
                        agg_sh.at[pl.ds(row0, ROWS_PER_TILE)])

        @pl.when(s == SC_SUBCORES - 1)
        def _init_tail():
            tail0 = SC_SUBCORES * ROWS_PER_TILE
            pltpu.sync_copy(zeros_hbm.at[pl.ds(tail0, ROWS_REMAINDER)],
                            agg_sh.at[pl.ds(tail0, ROWS_REMAINDER)])

        plsc.subcore_barrier()

        sems = (sm0, sm1)
        idxs = (idx0, idx1)

        def fire(k, b):
            pltpu.async_copy(m_hbm.at[pl.ds(base0 + k * CH, CH)],
                             rows.at[b], sems[b])
            pltpu.sync_copy(dst_hbm.at[pl.ds(base0 + k * CH, CH)], idxs[b])

        fire(0, 0)
        fire(1, 1)

        def group(g, carry):
            for b in range(2):
                k = 2 * g + b

                @pl.when(k < nch)
                def _drain():
                    pltpu.make_async_copy(m_hbm.at[pl.ds(0, CH)],
                                          rows.at[b], sems[b]).wait()
                    pltpu.sync_copy(rows.at[b], agg_sh.at[idxs[b]], add=True)

                    @pl.when(k + 2 < nch)
                    def _refill():
                        fire(k + 2, b)

            return carry

        lax.fori_loop(0, ngroup, group, 0)
        plsc.subcore_barrier()
        pltpu.sync_copy(agg_sh.at[pl.ds(row0, ROWS_PER_TILE)],
                        out_hbm.at[pl.ds(c * N + row0, ROWS_PER_TILE)])

        @pl.when(s == SC_SUBCORES - 1)
        def _out_tail():
            tail0 = SC_SUBCORES * ROWS_PER_TILE
            pltpu.sync_copy(agg_sh.at[pl.ds(tail0, ROWS_REMAINDER)],
                            out_hbm.at[pl.ds(c * N + tail0, ROWS_REMAINDER)])

    return scatter_add


EH = E // 2
_gather_half = _make_gather(EH)
_scatter2_full = _make_scatter2(E, EH)


def _stack_w(W):
    scale = 1.0 / np.sqrt(W.shape[1] * W.shape[2])
    return jnp.transpose(W, (2, 1, 0)) * scale


def _silu(v):
    return v * jax.nn.sigmoid(v)


def _tp_sum(x, attr_cols, W_ref):
    acc = None
    for a in range(A):
        d = jnp.dot(x, W_ref[a], preferred_element_type=jnp.float32)
        d = d * attr_cols[a]
        acc = d if acc is None else acc + d
    return acc


def _two_stage_body(n_x2, silu_last, residual, edge_bf16, attr_as_cols,
                    stage2_flat):
    def body(*refs):
        xs, refs = refs[:1 + n_x2], refs[1 + n_x2:]
        if attr_as_cols:
            attr_refs, refs = refs[:A], refs[A:]
            attr = [r[...] for r in attr_refs]
        else:
            attr_ref, refs = refs[0], refs[1:]
            av = attr_ref[...]
            attr = [av[:, a : a + 1] for a in range(A)]
        Wa_ref, ba_ref, Wb_ref, bb_ref, out_ref = refs
        if n_x2 >= 2:
            x2 = xs[1][...]
            for r in xs[2:]:
                x2 = x2 + r[...]
            x = jnp.concatenate([xs[0][...], x2], axis=-1)
        elif n_x2 == 1:
            x = jnp.concatenate([xs[0][...], xs[1][...]], axis=-1)
        else:
            x = xs[0][...]
        if edge_bf16:
            x = x.astype(jnp.bfloat16)
        h1 = _silu(_tp_sum(x, attr, Wa_ref) + ba_ref[...])
        if edge_bf16:
            h1 = h1.astype(jnp.bfloat16)
        if stage2_flat:
            y2 = jnp.concatenate([h1 * attr[a] for a in range(A)], axis=-1)
            o = jnp.dot(y2, Wb_ref[...],
                        preferred_element_type=jnp.float32) + bb_ref[...]
        else:
            o = _tp_sum(h1, attr, Wb_ref) + bb_ref[...]
        if silu_last:
            o = _silu(o)
        if residual:
            o = o + xs[0][...]
        out_ref[...] = o

    return body


def _emb_body(x_ref, attr_ref, W_ref, b_ref, out_ref):
    av = attr_ref[...]
    attr = [av[:, a : a + 1] for a in range(A)]
    out_ref[...] = _tp_sum(x_ref[...], attr, W_ref) + b_ref[...]


def _full_spec(shape):
    nd = len(shape)
    return pl.BlockSpec(shape, lambda i, _n=nd: (0,) * _n)


def _tp2_call(x1, x2s, attr, Wa, ba, Wb, bb, *, silu_last, residual, blk,
              edge_bf16=False, stage2_flat=False, attr_off=0):
    M = x1.shape[0]
    assert M % blk == 0
    n_x2 = len(x2s)
    Was = _stack_w(Wa)
    Wbs = _stack_w(Wb)
    if stage2_flat:
        Wbs = Wbs.reshape(-1, D)
    if edge_bf16:
        Was = Was.astype(jnp.bfloat16)
        Wbs = Wbs.astype(jnp.bfloat16)
    ba2 = ba.reshape(1, D)
    bb2 = bb.reshape(1, D)
    attr_as_cols = isinstance(attr, (list, tuple))
    attr_args = list(attr) if attr_as_cols else [attr]
    args = [x1] + [a for (a, _) in x2s] + attr_args + [Was, ba2, Wbs, bb2]
    in_specs = [pl.BlockSpec((blk, D), lambda i: (i, 0))]
    in_specs += [pl.BlockSpec((blk, D), lambda i, _o=off: (i + _o, 0))
                 for (_, off) in x2s]
    if attr_as_cols:
        in_specs += [pl.BlockSpec((blk, 1), lambda i: (i, 0))
                     for _ in range(A)]
    else:
        in_specs += [pl.BlockSpec((blk, A),
                                  lambda i, _ao=attr_off: (i + _ao, 0))]
    in_specs += [
        _full_spec(Was.shape),
        _full_spec((1, D)),
        _full_spec(Wbs.shape),
        _full_spec((1, D)),
    ]
    return pl.pallas_call(
        _two_stage_body(n_x2, silu_last, residual, edge_bf16, attr_as_cols,
                        stage2_flat),
        grid=(M // blk,),
        in_specs=in_specs,
        out_specs=pl.BlockSpec((blk, D), lambda i: (i, 0)),
        out_shape=jax.ShapeDtypeStruct((M, D), jnp.float32),
    )(*args)


def _emb_call(x, attr, W, b, *, blk):
    M = x.shape[0]
    Ws = _stack_w(W)
    b2 = b.reshape(1, D)
    return pl.pallas_call(
        _emb_body,
        grid=(M // blk,),
        in_specs=[
            pl.BlockSpec((blk, D), lambda i: (i, 0)),
            pl.BlockSpec((blk, A), lambda i: (i, 0)),
            _full_spec(Ws.shape),
            _full_spec((1, D)),
        ],
        out_specs=pl.BlockSpec((blk, D), lambda i: (i, 0)),
        out_shape=jax.ShapeDtypeStruct((M, D), jnp.float32),
    )(x, attr, Ws, b2)


def kernel(x, pos, edge_index, edge_attr, node_attr, batch, W_emb, b_emb,
           W_msg1_0, b_msg1_0, W_msg2_0, b_msg2_0, W_upd1_0, b_upd1_0,
           W_upd2_0, b_upd2_0, W_msg1_1, b_msg1_1, W_msg2_1, b_msg2_1,
           W_upd1_1, b_upd1_1, W_upd2_1, b_upd2_1, W_pre1, b_pre1,
           W_pre2, b_pre2):
    na = jnp.where(jnp.arange(A) == 0, 1.0, node_attr)
    h = _emb_call(x, na, W_emb, b_emb, blk=NODE_BLK)
    src = edge_index[0]
    dst = edge_index[1]
    pad = jnp.zeros((CH,), jnp.int32)
    dstA_p = jnp.concatenate([lax.slice(dst, (0,), (EH,)), pad])
    srcA_p = jnp.concatenate([lax.slice(src, (0,), (EH,)), pad])
    dstB_p = jnp.concatenate([lax.slice(dst, (EH,), (E,)), pad])
    srcB_p = jnp.concatenate([lax.slice(src, (EH,), (E,)), pad])
    zeros_nd = jnp.zeros((N, D), jnp.float32)
    layers = [
        (W_msg1_0, b_msg1_0, W_msg2_0, b_msg2_0, W_upd1_0, b_upd1_0, W_upd2_0, b_upd2_0),
        (W_msg1_1, b_msg1_1, W_msg2_1, b_msg2_1, W_upd1_1, b_upd1_1, W_upd2_1, b_upd2_1),
    ]
    for (Wm1, bm1, Wm2, bm2, Wu1, bu1, Wu2, bu2) in layers:
        hdA, hsA = _gather_half(h, dstA_p, srcA_p)
        hdB, hsB = _gather_half(h, dstB_p, srcB_p)
        m2A = _tp2_call(hdA, [(hsA, 0)], edge_attr, Wm1, bm1, Wm2, bm2,
                        silu_last=True, residual=False, blk=EDGE_BLK)
        m2B = _tp2_call(hdB, [(hsB, 0)], edge_attr, Wm1, bm1, Wm2, bm2,
                        silu_last=True, residual=False, blk=EDGE_BLK,
                        attr_off=EH // EDGE_BLK)
        agg2 = _scatter2_full(m2A, m2B, dst, zeros_nd)
        h = _tp2_call(h, [(agg2, 0), (agg2, N // NODE_BLK)], na,
                      Wu1, bu1, Wu2, bu2,
                      silu_last=False, residual=True, blk=NODE_BLK,
                      stage2_flat=True)
    h = _tp2_call(h, [], na, W_pre1, b_pre1, W_pre2, b_pre2,
                  silu_last=False, residual=False, blk=NODE_BLK,
                  stage2_flat=True)
    return h


# speedup vs baseline: 1.3556x; 1.0430x over previous
import functools

import jax
import jax.numpy as jnp
import numpy as np
from jax import lax
from jax.experimental import pallas as pl
from jax.experimental.pallas import tpu as pltpu
from jax.experimental.pallas import tpu_sc as plsc

N = 10000
E = 160000
D = 128
A = 4

NODE_BLK = 2000
EDGE_BLK = 2000

SC_CORES = 2
SC_SUBCORES = 16
NW = SC_CORES * SC_SUBCORES
CH = 128
ROWS_PER_TILE = (N // SC_SUBCORES) // 8 * 8
ROWS_REMAINDER = N - SC_SUBCORES * ROWS_PER_TILE

_sc_mesh = plsc.VectorSubcoreMesh(core_axis_name="c", subcore_axis_name="s")


def _make_gather(n_edges):
    n_chunk = n_edges // CH
    base_ch = n_chunk // NW
    extra_ch = n_chunk % NW
    idxbuf = (base_ch + 1) * CH
    nslot = 3
    ngroup = (base_ch + nslot) // nslot

    @functools.partial(
        pl.kernel,
        out_type=(jax.ShapeDtypeStruct((n_edges, D), jnp.float32),
                  jax.ShapeDtypeStruct((n_edges, D), jnp.float32)),
        mesh=_sc_mesh,
        scratch_types=[
            pltpu.VMEM((idxbuf,), jnp.int32),
            pltpu.VMEM((idxbuf,), jnp.int32),
            pltpu.VMEM((nslot, CH, D), jnp.float32),
            pltpu.VMEM((nslot, CH, D), jnp.float32),
        ] + [pltpu.SemaphoreType.DMA] * (2 * nslot),
    )
    def gather2(h_hbm, dst_hbm, src_hbm, od_hbm, os_hbm,
                idxd_all, idxs_all, rowd, rows, *sems_all):
        wid = lax.axis_index("s") * SC_CORES + lax.axis_index("c")
        nch = base_ch + (wid < extra_ch).astype(jnp.int32)
        start = wid * base_ch + jnp.minimum(wid, extra_ch)
        base0 = start * CH
        pltpu.sync_copy(dst_hbm.at[pl.ds(base0, idxbuf)], idxd_all)
        pltpu.sync_copy(src_hbm.at[pl.ds(base0, idxbuf)], idxs_all)
        semd = sems_all[:nslot]
        sems = sems_all[nslot:]

        def fire(k, b):
            pltpu.async_copy(h_hbm.at[idxd_all.at[pl.ds(k * CH, CH)]],
                             rowd.at[b], semd[b])
            pltpu.async_copy(h_hbm.at[idxs_all.at[pl.ds(k * CH, CH)]],
                             rows.at[b], sems[b])

        for b in range(nslot):
            fire(b, b)

        def group(g, carry):
            for b in range(nslot):
                k = nslot * g + b

                @pl.when(k < nch)
                def _drain():
                    pltpu.make_async_copy(od_hbm.at[pl.ds(0, CH)],
                                          rowd.at[b], semd[b]).wait()
                    pltpu.make_async_copy(od_hbm.at[pl.ds(0, CH)],
                                          rows.at[b], sems[b]).wait()
                    pltpu.sync_copy(rowd.at[b],
                                    od_hbm.at[pl.ds(base0 + k * CH, CH)])
                    pltpu.sync_copy(rows.at[b],
                                    os_hbm.at[pl.ds(base0 + k * CH, CH)])

                    @pl.when(k + nslot < nch)
                    def _refill():
                        fire(k + nslot, b)

            return carry

        lax.fori_loop(0, ngroup, group, 0)

    return gather2


def _make_scatter2(n_edges, n_a):
    n_chunk = n_edges // CH
    ch_a = n_a // CH
    base_ch = n_chunk // NW
    extra_ch = n_chunk % NW
    ngroup = (base_ch + 2) // 2

    @functools.partial(
        pl.kernel,
        out_type=jax.ShapeDtypeStruct((SC_CORES * N, D), jnp.float32),
        mesh=_sc_mesh,
        scratch_types=[
            pltpu.VMEM((CH,), jnp.int32),
            pltpu.VMEM((CH,), jnp.int32),
            pltpu.VMEM((2, CH, D), jnp.float32),
            pltpu.VMEM_SHARED((N, D), jnp.float32),
            pltpu.SemaphoreType.DMA,
            pltpu.SemaphoreType.DMA,
        ],
    )
    def scatter_add2(ma_hbm, mb_hbm, dst_hbm, zeros_hbm, out_hbm,
                     idx0, idx1, rows, agg_sh, sm0, sm1):
        c = lax.axis_index("c")
        s = lax.axis_index("s")
        wid = s * SC_CORES + c
        nch = base_ch + (wid < extra_ch).astype(jnp.int32)
        start = wid * base_ch + jnp.minimum(wid, extra_ch)
        base0 = start * CH

        row0 = s * ROWS_PER_TILE
        pltpu.sync_copy(zeros_hbm.at[pl.ds(row0, ROWS_PER_TILE)],
                        agg_sh.at[pl.ds(row0, ROWS_PER_TILE)])

        @pl.when(s == SC_SUBCORES - 1)
        def _init_tail():
            tail0 = SC_SUBCORES * ROWS_PER_TILE
            pltpu.sync_copy(zeros_hbm.at[pl.ds(tail0, ROWS_REMAINDER)],
                            agg_sh.at[pl.ds(tail0, ROWS_REMAINDER)])

        plsc.subcore_barrier()

        sems = (sm0, sm1)
        idxs = (idx0, idx1)

        def fire(k, b):
            kg = start + k

            @pl.when(kg < ch_a)
            def _fa():
                pltpu.async_copy(ma_hbm.at[pl.ds(kg * CH, CH)],
                                 rows.at[b], sems[b])

            @pl.when(kg >= ch_a)
            def _fb():
                pltpu.async_copy(mb_hbm.at[pl.ds((kg - ch_a) * CH, CH)],
                                 rows.at[b], sems[b])

            pltpu.sync_copy(dst_hbm.at[pl.ds(base0 + k * CH, CH)], idxs[b])

        fire(0, 0)
        fire(1, 1)

        def group(g, carry):
            for b in range(2):
                k = 2 * g + b

                @pl.when(k < nch)
                def _drain():
                    pltpu.make_async_copy(ma_hbm.at[pl.ds(0, CH)],
                                          rows.at[b], sems[b]).wait()
                    pltpu.sync_copy(rows.at[b], agg_sh.at[idxs[b]], add=True)

                    @pl.when(k + 2 < nch)
                    def _refill():
                        fire(k + 2, b)

            return carry

        lax.fori_loop(0, ngroup, group, 0)
        plsc.subcore_barrier()
        pltpu.sync_copy(agg_sh.at[pl.ds(row0, ROWS_PER_TILE)],
                        out_hbm.at[pl.ds(c * N + row0, ROWS_PER_TILE)])

        @pl.when(s == SC_SUBCORES - 1)
        def _out_tail():
            tail0 = SC_SUBCORES * ROWS_PER_TILE
            pltpu.sync_copy(agg_sh.at[pl.ds(tail0, ROWS_REMAINDER)],
                            out_hbm.at[pl.ds(c * N + tail0, ROWS_REMAINDER)])

    return scatter_add2


def _make_scatter(n_edges):
    n_chunk = n_edges // CH
    base_ch = n_chunk // NW
    extra_ch = n_chunk % NW
    ngroup = (base_ch + 2) // 2

    @functools.partial(
        pl.kernel,
        out_type=jax.ShapeDtypeStruct((SC_CORES * N, D), jnp.float32),
        mesh=_sc_mesh,
        scratch_types=[
            pltpu.VMEM((CH,), jnp.int32),
            pltpu.VMEM((CH,), jnp.int32),
            pltpu.VMEM((2, CH, D), jnp.float32),
            pltpu.VMEM_SHARED((N, D), jnp.float32),
            pltpu.SemaphoreType.DMA,
            pltpu.SemaphoreType.DMA,
        ],
    )
    def scatter_add(m_hbm, dst_hbm, zeros_hbm, out_hbm,
                    idx0, idx1, rows, agg_sh, sm0, sm1):
        c = lax.axis_index("c")
        s = lax.axis_index("s")
        wid = s * SC_CORES + c
        nch = base_ch + (wid < extra_ch).astype(jnp.int32)
        start = wid * base_ch + jnp.minimum(wid, extra_ch)
        base0 = start * CH

        row0 = s * ROWS_PER_TILE
        pltpu.sync_copy(zeros_hbm.at[pl.ds(row0, ROWS_PER_TILE)],
                        agg_sh.at[pl.ds(row0, ROWS_PER_TILE)])

        @pl.when(s == SC_SUBCORES - 1)
        def _init_tail():
            tail0 = SC_SUBCORES * ROWS_PER_TILE
            pltpu.sync_copy(zeros_hbm.at[pl.ds(tail0, ROWS_REMAINDER)],
                            agg_sh.at[pl.ds(tail0, ROWS_REMAINDER)])

        plsc.subcore_barrier()

        sems = (sm0, sm1)
        idxs = (idx0, idx1)

        def fire(k, b):
            pltpu.async_copy(m_hbm.at[pl.ds(base0 + k * CH, CH)],
                             rows.at[b], sems[b])
            pltpu.sync_copy(dst_hbm.at[pl.ds(base0 + k * CH, CH)], idxs[b])

        fire(0, 0)
        fire(1, 1)

        def group(g, carry):
            for b in range(2):
                k = 2 * g + b

                @pl.when(k < nch)
                def _drain():
                    pltpu.make_async_copy(m_hbm.at[pl.ds(0, CH)],
                                          rows.at[b], sems[b]).wait()
                    pltpu.sync_copy(rows.at[b], agg_sh.at[idxs[b]], add=True)

                    @pl.when(k + 2 < nch)
                    def _refill():
                        fire(k + 2, b)

            return carry

        lax.fori_loop(0, ngroup, group, 0)
        plsc.subcore_barrier()
        pltpu.sync_copy(agg_sh.at[pl.ds(row0, ROWS_PER_TILE)],
                        out_hbm.at[pl.ds(c * N + row0, ROWS_PER_TILE)])

        @pl.when(s == SC_SUBCORES - 1)
        def _out_tail():
            tail0 = SC_SUBCORES * ROWS_PER_TILE
            pltpu.sync_copy(agg_sh.at[pl.ds(tail0, ROWS_REMAINDER)],
                            out_hbm.at[pl.ds(c * N + tail0, ROWS_REMAINDER)])

    return scatter_add


EA = 64000
EB = E - EA
_gather_a = _make_gather(EA)
_gather_b = _make_gather(EB)
_scatter_a = _make_scatter(EA)
_scatter_b = _make_scatter(EB)


def _stack_w(W):
    scale = 1.0 / np.sqrt(W.shape[1] * W.shape[2])
    return jnp.transpose(W, (2, 1, 0)) * scale


def _silu(v):
    return v * jax.nn.sigmoid(v)


def _tp_sum(x, attr_cols, W_ref):
    acc = None
    for a in range(A):
        d = jnp.dot(x, W_ref[a], preferred_element_type=jnp.float32)
        d = d * attr_cols[a]
        acc = d if acc is None else acc + d
    return acc


def _two_stage_body(n_x2, silu_last, residual, edge_bf16, attr_as_cols,
                    stage2_flat):
    def body(*refs):
        xs, refs = refs[:1 + n_x2], refs[1 + n_x2:]
        if attr_as_cols:
            attr_refs, refs = refs[:A], refs[A:]
            attr = [r[...] for r in attr_refs]
        else:
            attr_ref, refs = refs[0], refs[1:]
            av = attr_ref[...]
            attr = [av[:, a : a + 1] for a in range(A)]
        Wa_ref, ba_ref, Wb_ref, bb_ref, out_ref = refs
        if n_x2 >= 2:
            x2 = xs[1][...]
            for r in xs[2:]:
                x2 = x2 + r[...]
            x = jnp.concatenate([xs[0][...], x2], axis=-1)
        elif n_x2 == 1:
            x = jnp.concatenate([xs[0][...], xs[1][...]], axis=-1)
        else:
            x = xs[0][...]
        if edge_bf16:
            x = x.astype(jnp.bfloat16)
        h1 = _silu(_tp_sum(x, attr, Wa_ref) + ba_ref[...])
        if edge_bf16:
            h1 = h1.astype(jnp.bfloat16)
        if stage2_flat:
            y2 = jnp.concatenate([h1 * attr[a] for a in range(A)], axis=-1)
            o = jnp.dot(y2, Wb_ref[...],
                        preferred_element_type=jnp.float32) + bb_ref[...]
        else:
            o = _tp_sum(h1, attr, Wb_ref) + bb_ref[...]
        if silu_last:
            o = _silu(o)
        if residual:
            o = o + xs[0][...]
        out_ref[...] = o

    return body


def _emb_body(x_ref, attr_ref, W_ref, b_ref, out_ref):
    av = attr_ref[...]
    attr = [av[:, a : a + 1] for a in range(A)]
    out_ref[...] = _tp_sum(x_ref[...], attr, W_ref) + b_ref[...]


def _full_spec(shape):
    nd = len(shape)
    return pl.BlockSpec(shape, lambda i, _n=nd: (0,) * _n)


def _tp2_call(x1, x2s, attr, Wa, ba, Wb, bb, *, silu_last, residual, blk,
              edge_bf16=False, stage2_flat=False, attr_off=0):
    M = x1.shape[0]
    assert M % blk == 0
    n_x2 = len(x2s)
    Was = _stack_w(Wa)
    Wbs = _stack_w(Wb)
    if stage2_flat:
        Wbs = Wbs.reshape(-1, D)
    if edge_bf16:
        Was = Was.astype(jnp.bfloat16)
        Wbs = Wbs.astype(jnp.bfloat16)
    ba2 = ba.reshape(1, D)
    bb2 = bb.reshape(1, D)
    attr_as_cols = isinstance(attr, (list, tuple))
    attr_args = list(attr) if attr_as_cols else [attr]
    args = [x1] + [a for (a, _) in x2s] + attr_args + [Was, ba2, Wbs, bb2]
    in_specs = [pl.BlockSpec((blk, D), lambda i: (i, 0))]
    in_specs += [pl.BlockSpec((blk, D), lambda i, _o=off: (i + _o, 0))
                 for (_, off) in x2s]
    if attr_as_cols:
        in_specs += [pl.BlockSpec((blk, 1), lambda i: (i, 0))
                     for _ in range(A)]
    else:
        in_specs += [pl.BlockSpec((blk, A),
                                  lambda i, _ao=attr_off: (i + _ao, 0))]
    in_specs += [
        _full_spec(Was.shape),
        _full_spec((1, D)),
        _full_spec(Wbs.shape),
        _full_spec((1, D)),
    ]
    return pl.pallas_call(
        _two_stage_body(n_x2, silu_last, residual, edge_bf16, attr_as_cols,
                        stage2_flat),
        grid=(M // blk,),
        in_specs=in_specs,
        out_specs=pl.BlockSpec((blk, D), lambda i: (i, 0)),
        out_shape=jax.ShapeDtypeStruct((M, D), jnp.float32),
    )(*args)


def _emb_call(x, attr, W, b, *, blk):
    M = x.shape[0]
    Ws = _stack_w(W)
    b2 = b.reshape(1, D)
    return pl.pallas_call(
        _emb_body,
        grid=(M // blk,),
        in_specs=[
            pl.BlockSpec((blk, D), lambda i: (i, 0)),
            pl.BlockSpec((blk, A), lambda i: (i, 0)),
            _full_spec(Ws.shape),
            _full_spec((1, D)),
        ],
        out_specs=pl.BlockSpec((blk, D), lambda i: (i, 0)),
        out_shape=jax.ShapeDtypeStruct((M, D), jnp.float32),
    )(x, attr, Ws, b2)


def kernel(x, pos, edge_index, edge_attr, node_attr, batch, W_emb, b_emb,
           W_msg1_0, b_msg1_0, W_msg2_0, b_msg2_0, W_upd1_0, b_upd1_0,
           W_upd2_0, b_upd2_0, W_msg1_1, b_msg1_1, W_msg2_1, b_msg2_1,
           W_upd1_1, b_upd1_1, W_upd2_1, b_upd2_1, W_pre1, b_pre1,
           W_pre2, b_pre2):
    na = jnp.where(jnp.arange(A) == 0, 1.0, node_attr)
    h = _emb_call(x, na, W_emb, b_emb, blk=NODE_BLK)
    src = edge_index[0]
    dst = edge_index[1]
    pad = jnp.zeros((CH,), jnp.int32)
    dstA = lax.slice(dst, (0,), (EA,))
    srcA_p = jnp.concatenate([lax.slice(src, (0,), (EA,)), pad])
    dstA_p = jnp.concatenate([dstA, pad])
    dstB = lax.slice(dst, (EA,), (E,))
    srcB_p = jnp.concatenate([lax.slice(src, (EA,), (E,)), pad])
    dstB_p = jnp.concatenate([dstB, pad])
    zeros_nd = jnp.zeros((N, D), jnp.float32)
    layers = [
        (W_msg1_0, b_msg1_0, W_msg2_0, b_msg2_0, W_upd1_0, b_upd1_0, W_upd2_0, b_upd2_0),
        (W_msg1_1, b_msg1_1, W_msg2_1, b_msg2_1, W_upd1_1, b_upd1_1, W_upd2_1, b_upd2_1),
    ]
    for (Wm1, bm1, Wm2, bm2, Wu1, bu1, Wu2, bu2) in layers:
        hdA, hsA = _gather_a(h, dstA_p, srcA_p)
        hdB, hsB = _gather_b(h, dstB_p, srcB_p)
        m2A = _tp2_call(hdA, [(hsA, 0)], edge_attr, Wm1, bm1, Wm2, bm2,
                        silu_last=True, residual=False, blk=EDGE_BLK)
        aggA = _scatter_a(m2A, dstA, zeros_nd)
        m2B = _tp2_call(hdB, [(hsB, 0)], edge_attr, Wm1, bm1, Wm2, bm2,
                        silu_last=True, residual=False, blk=EDGE_BLK,
                        attr_off=EA // EDGE_BLK)
        aggB = _scatter_b(m2B, dstB, zeros_nd)
        noff = N // NODE_BLK
        h = _tp2_call(h, [(aggA, 0), (aggA, noff), (aggB, 0), (aggB, noff)],
                      na, Wu1, bu1, Wu2, bu2,
                      silu_last=False, residual=True, blk=NODE_BLK,
                      stage2_flat=True)
    h = _tp2_call(h, [], na, W_pre1, b_pre1, W_pre2, b_pre2,
                  silu_last=False, residual=False, blk=NODE_BLK,
                  stage2_flat=True)
    return h


# speedup vs baseline: 1.3670x; 1.0085x over previous
import functools

import jax
import jax.numpy as jnp
import numpy as np
from jax import lax
from jax.experimental import pallas as pl
from jax.experimental.pallas import tpu as pltpu
from jax.experimental.pallas import tpu_sc as plsc

N = 10000
E = 160000
D = 128
A = 4

NODE_BLK = 2000
EDGE_BLK = 2000

SC_CORES = 2
SC_SUBCORES = 16
NW = SC_CORES * SC_SUBCORES
CH = 128
ROWS_PER_TILE = (N // SC_SUBCORES) // 8 * 8
ROWS_REMAINDER = N - SC_SUBCORES * ROWS_PER_TILE

_sc_mesh = plsc.VectorSubcoreMesh(core_axis_name="c", subcore_axis_name="s")


def _make_gather(n_edges):
    n_chunk = n_edges // CH
    base_ch = n_chunk // NW
    extra_ch = n_chunk % NW
    idxbuf = (base_ch + 1) * CH
    nslot = 3
    ngroup = (base_ch + nslot) // nslot

    @functools.partial(
        pl.kernel,
        out_type=(jax.ShapeDtypeStruct((n_edges, D), jnp.float32),
                  jax.ShapeDtypeStruct((n_edges, D), jnp.float32)),
        mesh=_sc_mesh,
        scratch_types=[
            pltpu.VMEM((idxbuf,), jnp.int32),
            pltpu.VMEM((idxbuf,), jnp.int32),
            pltpu.VMEM((nslot, CH, D), jnp.float32),
            pltpu.VMEM((nslot, CH, D), jnp.float32),
        ] + [pltpu.SemaphoreType.DMA] * (2 * nslot),
    )
    def gather2(h_hbm, dst_hbm, src_hbm, od_hbm, os_hbm,
                idxd_all, idxs_all, rowd, rows, *sems_all):
        wid = lax.axis_index("s") * SC_CORES + lax.axis_index("c")
        nch = base_ch + (wid < extra_ch).astype(jnp.int32)
        start = wid * base_ch + jnp.minimum(wid, extra_ch)
        base0 = start * CH
        pltpu.sync_copy(dst_hbm.at[pl.ds(base0, idxbuf)], idxd_all)
        pltpu.sync_copy(src_hbm.at[pl.ds(base0, idxbuf)], idxs_all)
        semd = sems_all[:nslot]
        sems = sems_all[nslot:]

        def fire(k, b):
            pltpu.async_copy(h_hbm.at[idxd_all.at[pl.ds(k * CH, CH)]],
                             rowd.at[b], semd[b])
            pltpu.async_copy(h_hbm.at[idxs_all.at[pl.ds(k * CH, CH)]],
                             rows.at[b], sems[b])

        for b in range(nslot):
            fire(b, b)

        def group(g, carry):
            for b in range(nslot):
                k = nslot * g + b

                @pl.when(k < nch)
                def _drain():
                    pltpu.make_async_copy(od_hbm.at[pl.ds(0, CH)],
                                          rowd.at[b], semd[b]).wait()
                    pltpu.make_async_copy(od_hbm.at[pl.ds(0, CH)],
                                          rows.at[b], sems[b]).wait()
                    pltpu.sync_copy(rowd.at[b],
                                    od_hbm.at[pl.ds(base0 + k * CH, CH)])
                    pltpu.sync_copy(rows.at[b],
                                    os_hbm.at[pl.ds(base0 + k * CH, CH)])

                    @pl.when(k + nslot < nch)
                    def _refill():
                        fire(k + nslot, b)

            return carry

        lax.fori_loop(0, ngroup, group, 0)

    return gather2


def _make_scatter2(n_edges, n_a):
    n_chunk = n_edges // CH
    ch_a = n_a // CH
    base_ch = n_chunk // NW
    extra_ch = n_chunk % NW
    ngroup = (base_ch + 2) // 2

    @functools.partial(
        pl.kernel,
        out_type=jax.ShapeDtypeStruct((SC_CORES * N, D), jnp.float32),
        mesh=_sc_mesh,
        scratch_types=[
            pltpu.VMEM((CH,), jnp.int32),
            pltpu.VMEM((CH,), jnp.int32),
            pltpu.VMEM((2, CH, D), jnp.float32),
            pltpu.VMEM_SHARED((N, D), jnp.float32),
            pltpu.SemaphoreType.DMA,
            pltpu.SemaphoreType.DMA,
        ],
    )
    def scatter_add2(ma_hbm, mb_hbm, dst_hbm, zeros_hbm, out_hbm,
                     idx0, idx1, rows, agg_sh, sm0, sm1):
        c = lax.axis_index("c")
        s = lax.axis_index("s")
        wid = s * SC_CORES + c
        nch = base_ch + (wid < extra_ch).astype(jnp.int32)
        start = wid * base_ch + jnp.minimum(wid, extra_ch)
        base0 = start * CH

        row0 = s * ROWS_PER_TILE
        pltpu.sync_copy(zeros_hbm.at[pl.ds(row0, ROWS_PER_TILE)],
                        agg_sh.at[pl.ds(row0, ROWS_PER_TILE)])

        @pl.when(s == SC_SUBCORES - 1)
        def _init_tail():
            tail0 = SC_SUBCORES * ROWS_PER_TILE
            pltpu.sync_copy(zeros_hbm.at[pl.ds(tail0, ROWS_REMAINDER)],
                            agg_sh.at[pl.ds(tail0, ROWS_REMAINDER)])

        plsc.subcore_barrier()

        sems = (sm0, sm1)
        idxs = (idx0, idx1)

        def fire(k, b):
            kg = start + k

            @pl.when(kg < ch_a)
            def _fa():
                pltpu.async_copy(ma_hbm.at[pl.ds(kg * CH, CH)],
                                 rows.at[b], sems[b])

            @pl.when(kg >= ch_a)
            def _fb():
                pltpu.async_copy(mb_hbm.at[pl.ds((kg - ch_a) * CH, CH)],
                                 rows.at[b], sems[b])

            pltpu.sync_copy(dst_hbm.at[pl.ds(base0 + k * CH, CH)], idxs[b])

        fire(0, 0)
        fire(1, 1)

        def group(g, carry):
            for b in range(2):
                k = 2 * g + b

                @pl.when(k < nch)
                def _drain():
                    pltpu.make_async_copy(ma_hbm.at[pl.ds(0, CH)],
                                          rows.at[b], sems[b]).wait()
                    pltpu.sync_copy(rows.at[b], agg_sh.at[idxs[b]], add=True)

                    @pl.when(k + 2 < nch)
                    def _refill():
                        fire(k + 2, b)

            return carry

        lax.fori_loop(0, ngroup, group, 0)
        plsc.subcore_barrier()
        pltpu.sync_copy(agg_sh.at[pl.ds(row0, ROWS_PER_TILE)],
                        out_hbm.at[pl.ds(c * N + row0, ROWS_PER_TILE)])

        @pl.when(s == SC_SUBCORES - 1)
        def _out_tail():
            tail0 = SC_SUBCORES * ROWS_PER_TILE
            pltpu.sync_copy(agg_sh.at[pl.ds(tail0, ROWS_REMAINDER)],
                            out_hbm.at[pl.ds(c * N + tail0, ROWS_REMAINDER)])

    return scatter_add2


def _make_scatter(n_edges):
    n_chunk = n_edges // CH
    base_ch = n_chunk // NW
    extra_ch = n_chunk % NW
    ngroup = (base_ch + 2) // 2

    @functools.partial(
        pl.kernel,
        out_type=jax.ShapeDtypeStruct((SC_CORES * N, D), jnp.float32),
        mesh=_sc_mesh,
        scratch_types=[
            pltpu.VMEM((CH,), jnp.int32),
            pltpu.VMEM((CH,), jnp.int32),
            pltpu.VMEM((2, CH, D), jnp.float32),
            pltpu.VMEM_SHARED((N, D), jnp.float32),
            pltpu.SemaphoreType.DMA,
            pltpu.SemaphoreType.DMA,
        ],
    )
    def scatter_add(m_hbm, dst_hbm, zeros_hbm, out_hbm,
                    idx0, idx1, rows, agg_sh, sm0, sm1):
        c = lax.axis_index("c")
        s = lax.axis_index("s")
        wid = s * SC_CORES + c
        nch = base_ch + (wid < extra_ch).astype(jnp.int32)
        start = wid * base_ch + jnp.minimum(wid, extra_ch)
        base0 = start * CH

        row0 = s * ROWS_PER_TILE
        pltpu.sync_copy(zeros_hbm.at[pl.ds(row0, ROWS_PER_TILE)],
                        agg_sh.at[pl.ds(row0, ROWS_PER_TILE)])

        @pl.when(s == SC_SUBCORES - 1)
        def _init_tail():
            tail0 = SC_SUBCORES * ROWS_PER_TILE
            pltpu.sync_copy(zeros_hbm.at[pl.ds(tail0, ROWS_REMAINDER)],
                            agg_sh.at[pl.ds(tail0, ROWS_REMAINDER)])

        plsc.subcore_barrier()

        sems = (sm0, sm1)
        idxs = (idx0, idx1)

        def fire(k, b):
            pltpu.async_copy(m_hbm.at[pl.ds(base0 + k * CH, CH)],
                             rows.at[b], sems[b])
            pltpu.sync_copy(dst_hbm.at[pl.ds(base0 + k * CH, CH)], idxs[b])

        fire(0, 0)
        fire(1, 1)

        def group(g, carry):
            for b in range(2):
                k = 2 * g + b

                @pl.when(k < nch)
                def _drain():
                    pltpu.make_async_copy(m_hbm.at[pl.ds(0, CH)],
                                          rows.at[b], sems[b]).wait()
                    pltpu.sync_copy(rows.at[b], agg_sh.at[idxs[b]], add=True)

                    @pl.when(k + 2 < nch)
                    def _refill():
                        fire(k + 2, b)

            return carry

        lax.fori_loop(0, ngroup, group, 0)
        plsc.subcore_barrier()
        pltpu.sync_copy(agg_sh.at[pl.ds(row0, ROWS_PER_TILE)],
                        out_hbm.at[pl.ds(c * N + row0, ROWS_PER_TILE)])

        @pl.when(s == SC_SUBCORES - 1)
        def _out_tail():
            tail0 = SC_SUBCORES * ROWS_PER_TILE
            pltpu.sync_copy(agg_sh.at[pl.ds(tail0, ROWS_REMAINDER)],
                            out_hbm.at[pl.ds(c * N + tail0, ROWS_REMAINDER)])

    return scatter_add


EA = 96000
EB = E - EA
_gather_a = _make_gather(EA)
_gather_b = _make_gather(EB)
_scatter_a = _make_scatter(EA)
_scatter_b = _make_scatter(EB)


def _stack_w(W):
    scale = 1.0 / np.sqrt(W.shape[1] * W.shape[2])
    return jnp.transpose(W, (2, 1, 0)) * scale


def _silu(v):
    return v * jax.nn.sigmoid(v)


def _tp_sum(x, attr_cols, W_ref):
    acc = None
    for a in range(A):
        d = jnp.dot(x, W_ref[a], preferred_element_type=jnp.float32)
        d = d * attr_cols[a]
        acc = d if acc is None else acc + d
    return acc


def _two_stage_body(n_x2, silu_last, residual, edge_bf16, attr_as_cols,
                    stage2_flat):
    def body(*refs):
        xs, refs = refs[:1 + n_x2], refs[1 + n_x2:]
        if attr_as_cols:
            attr_refs, refs = refs[:A], refs[A:]
            attr = [r[...] for r in attr_refs]
        else:
            attr_ref, refs = refs[0], refs[1:]
            av = attr_ref[...]
            attr = [av[:, a : a + 1] for a in range(A)]
        Wa_ref, ba_ref, Wb_ref, bb_ref, out_ref = refs
        if n_x2 >= 2:
            x2 = xs[1][...]
            for r in xs[2:]:
                x2 = x2 + r[...]
            x = jnp.concatenate([xs[0][...], x2], axis=-1)
        elif n_x2 == 1:
            x = jnp.concatenate([xs[0][...], xs[1][...]], axis=-1)
        else:
            x = xs[0][...]
        if edge_bf16:
            x = x.astype(jnp.bfloat16)
        h1 = _silu(_tp_sum(x, attr, Wa_ref) + ba_ref[...])
        if edge_bf16:
            h1 = h1.astype(jnp.bfloat16)
        if stage2_flat:
            y2 = jnp.concatenate([h1 * attr[a] for a in range(A)], axis=-1)
            o = jnp.dot(y2, Wb_ref[...],
                        preferred_element_type=jnp.float32) + bb_ref[...]
        else:
            o = _tp_sum(h1, attr, Wb_ref) + bb_ref[...]
        if silu_last:
            o = _silu(o)
        if residual:
            o = o + xs[0][...]
        out_ref[...] = o

    return body


def _emb_body(x_ref, attr_ref, W_ref, b_ref, out_ref):
    av = attr_ref[...]
    attr = [av[:, a : a + 1] for a in range(A)]
    out_ref[...] = _tp_sum(x_ref[...], attr, W_ref) + b_ref[...]


def _full_spec(shape):
    nd = len(shape)
    return pl.BlockSpec(shape, lambda i, _n=nd: (0,) * _n)


def _tp2_call(x1, x2s, attr, Wa, ba, Wb, bb, *, silu_last, residual, blk,
              edge_bf16=False, stage2_flat=False, attr_off=0):
    M = x1.shape[0]
    assert M % blk == 0
    n_x2 = len(x2s)
    Was = _stack_w(Wa)
    Wbs = _stack_w(Wb)
    if stage2_flat:
        Wbs = Wbs.reshape(-1, D)
    if edge_bf16:
        Was = Was.astype(jnp.bfloat16)
        Wbs = Wbs.astype(jnp.bfloat16)
    ba2 = ba.reshape(1, D)
    bb2 = bb.reshape(1, D)
    attr_as_cols = isinstance(attr, (list, tuple))
    attr_args = list(attr) if attr_as_cols else [attr]
    args = [x1] + [a for (a, _) in x2s] + attr_args + [Was, ba2, Wbs, bb2]
    in_specs = [pl.BlockSpec((blk, D), lambda i: (i, 0))]
    in_specs += [pl.BlockSpec((blk, D), lambda i, _o=off: (i + _o, 0))
                 for (_, off) in x2s]
    if attr_as_cols:
        in_specs += [pl.BlockSpec((blk, 1), lambda i: (i, 0))
                     for _ in range(A)]
    else:
        in_specs += [pl.BlockSpec((blk, A),
                                  lambda i, _ao=attr_off: (i + _ao, 0))]
    in_specs += [
        _full_spec(Was.shape),
        _full_spec((1, D)),
        _full_spec(Wbs.shape),
        _full_spec((1, D)),
    ]
    return pl.pallas_call(
        _two_stage_body(n_x2, silu_last, residual, edge_bf16, attr_as_cols,
                        stage2_flat),
        grid=(M // blk,),
        in_specs=in_specs,
        out_specs=pl.BlockSpec((blk, D), lambda i: (i, 0)),
        out_shape=jax.ShapeDtypeStruct((M, D), jnp.float32),
    )(*args)


def _emb_call(x, attr, W, b, *, blk):
    M = x.shape[0]
    Ws = _stack_w(W)
    b2 = b.reshape(1, D)
    return pl.pallas_call(
        _emb_body,
        grid=(M // blk,),
        in_specs=[
            pl.BlockSpec((blk, D), lambda i: (i, 0)),
            pl.BlockSpec((blk, A), lambda i: (i, 0)),
            _full_spec(Ws.shape),
            _full_spec((1, D)),
        ],
        out_specs=pl.BlockSpec((blk, D), lambda i: (i, 0)),
        out_shape=jax.ShapeDtypeStruct((M, D), jnp.float32),
    )(x, attr, Ws, b2)


def kernel(x, pos, edge_index, edge_attr, node_attr, batch, W_emb, b_emb,
           W_msg1_0, b_msg1_0, W_msg2_0, b_msg2_0, W_upd1_0, b_upd1_0,
           W_upd2_0, b_upd2_0, W_msg1_1, b_msg1_1, W_msg2_1, b_msg2_1,
           W_upd1_1, b_upd1_1, W_upd2_1, b_upd2_1, W_pre1, b_pre1,
           W_pre2, b_pre2):
    na = jnp.where(jnp.arange(A) == 0, 1.0, node_attr)
    h = _emb_call(x, na, W_emb, b_emb, blk=NODE_BLK)
    src = edge_index[0]
    dst = edge_index[1]
    pad = jnp.zeros((CH,), jnp.int32)
    dstA = lax.slice(dst, (0,), (EA,))
    srcA_p = jnp.concatenate([lax.slice(src, (0,), (EA,)), pad])
    dstA_p = jnp.concatenate([dstA, pad])
    dstB = lax.slice(dst, (EA,), (E,))
    srcB_p = jnp.concatenate([lax.slice(src, (EA,), (E,)), pad])
    dstB_p = jnp.concatenate([dstB, pad])
    zeros_nd = jnp.zeros((N, D), jnp.float32)
    layers = [
        (W_msg1_0, b_msg1_0, W_msg2_0, b_msg2_0, W_upd1_0, b_upd1_0, W_upd2_0, b_upd2_0),
        (W_msg1_1, b_msg1_1, W_msg2_1, b_msg2_1, W_upd1_1, b_upd1_1, W_upd2_1, b_upd2_1),
    ]
    for (Wm1, bm1, Wm2, bm2, Wu1, bu1, Wu2, bu2) in layers:
        hdA, hsA = _gather_a(h, dstA_p, srcA_p)
        hdB, hsB = _gather_b(h, dstB_p, srcB_p)
        m2A = _tp2_call(hdA, [(hsA, 0)], edge_attr, Wm1, bm1, Wm2, bm2,
                        silu_last=True, residual=False, blk=EDGE_BLK)
        aggA = _scatter_a(m2A, dstA, zeros_nd)
        m2B = _tp2_call(hdB, [(hsB, 0)], edge_attr, Wm1, bm1, Wm2, bm2,
                        silu_last=True, residual=False, blk=EDGE_BLK,
                        attr_off=EA // EDGE_BLK)
        aggB = _scatter_b(m2B, dstB, zeros_nd)
        noff = N // NODE_BLK
        h = _tp2_call(h, [(aggA, 0), (aggA, noff), (aggB, 0), (aggB, noff)],
                      na, Wu1, bu1, Wu2, bu2,
                      silu_last=False, residual=True, blk=NODE_BLK,
                      stage2_flat=True)
    h = _tp2_call(h, [], na, W_pre1, b_pre1, W_pre2, b_pre2,
                  silu_last=False, residual=False, blk=NODE_BLK,
                  stage2_flat=True)
    return h
